# baseline (speedup 1.0000x reference)
import functools

import jax
import jax.numpy as jnp
from jax import lax
from jax.experimental import pallas as pl
from jax.experimental.pallas import tpu as pltpu
from jax.experimental.pallas import tpu_sc as plsc

N = 10000
E = 320000
D_NODE = 128
D_EDGE = 16
D1 = 256
D2 = 128
A = 8
LOG_SIG_MAX = 2.0
LOG_SIG_MIN = -20.0

NCORES = 2
NSUB = 16
F1 = D1 // NCORES
CHUNK = 80
EDGES_PER_TILE = E // NSUB
EDGES_PER_TILE_B = E // (NCORES * NSUB)
ROWS_MAIN = 640
ROWS_LAST = N - (NSUB - 1) * ROWS_MAIN
CCHUNK = 80

EB = 512
NB = 1000


def _bdot(a, b):
    return jnp.dot(a.astype(jnp.bfloat16), b.astype(jnp.bfloat16),
                   preferred_element_type=jnp.float32)


def _edge_mlp_body(ef, We1, be1, We2, be2, e1o, e2o, se1o, se2o, acc1, acc2):
    i = pl.program_id(0)

    @pl.when(i == 0)
    def _():
        acc1[...] = jnp.zeros_like(acc1)
        acc2[...] = jnp.zeros_like(acc2)

    h = jnp.maximum(
        _bdot(ef[...], We1[...]) + be1[...], 0.0)
    e1o[...] = h
    z = jnp.maximum(
        _bdot(h, We2[...]) + be2[...], 0.0)
    e2o[...] = z
    acc1[...] += jnp.sum(h, axis=0, keepdims=True)
    acc2[...] += jnp.sum(z, axis=0, keepdims=True)

    @pl.when(i == pl.num_programs(0) - 1)
    def _():
        se1o[...] = acc1[...]
        se2o[...] = acc2[...]


def _edge_mlp(ef, We1, be1, We2, be2):
    grid = E // EB
    return pl.pallas_call(
        _edge_mlp_body,
        grid=(grid,),
        in_specs=[
            pl.BlockSpec((EB, D_EDGE), lambda i: (i, 0)),
            pl.BlockSpec((D_EDGE, D1), lambda i: (0, 0)),
            pl.BlockSpec((1, D1), lambda i: (0, 0)),
            pl.BlockSpec((D1, D2), lambda i: (0, 0)),
            pl.BlockSpec((1, D2), lambda i: (0, 0)),
        ],
        out_specs=[
            pl.BlockSpec((EB, D1), lambda i: (i, 0)),
            pl.BlockSpec((EB, D2), lambda i: (i, 0)),
            pl.BlockSpec((1, D1), lambda i: (0, 0)),
            pl.BlockSpec((1, D2), lambda i: (0, 0)),
        ],
        out_shape=[
            jax.ShapeDtypeStruct((E, D1), jnp.float32),
            jax.ShapeDtypeStruct((E, D2), jnp.float32),
            jax.ShapeDtypeStruct((1, D1), jnp.float32),
            jax.ShapeDtypeStruct((1, D2), jnp.float32),
        ],
        scratch_shapes=[
            pltpu.VMEM((1, D1), jnp.float32),
            pltpu.VMEM((1, D2), jnp.float32),
        ],
        compiler_params=pltpu.CompilerParams(
            dimension_semantics=("arbitrary",)),
    )(ef, We1, be1, We2, be2)


def _zero2d(ref, rows, width):
    zv = jnp.zeros((16,), jnp.float32)

    def row(r, _):
        def col(cc, _):
            ref[r, pl.ds(cc * 16, 16)] = zv
            return 0
        lax.fori_loop(0, width // 16, col, 0)
        return 0
    lax.fori_loop(0, rows, row, 0)


G = 4
NG_A = (E // NSUB) // CHUNK // G
REM_A = (E // NSUB) // CHUNK - NG_A * G
NG_B = EDGES_PER_TILE_B // CHUNK // G
REM_B = EDGES_PER_TILE_B // CHUNK - NG_B * G


def _seg_body(e1h, e2h, idxh, s1o, s2o, acc,
              b0, b1, b2, b3, i0, i1, i2, i3,
              g0, g1, g2, g3, s0, s1, s2, s3):
    cid = lax.axis_index("c")
    sid = lax.axis_index("s")
    fb1 = cid * F1
    tbase = sid * ROWS_MAIN
    nck = jnp.where(sid == NSUB - 1, ROWS_LAST // CHUNK, ROWS_MAIN // CHUNK)
    bufs = (b0, b1, b2, b3)
    idxs = (i0, i1, i2, i3)
    gsems = (g0, g1, g2, g3)
    ssems = (s0, s1, s2, s3)

    def scat(j):
        return pltpu.make_async_copy(bufs[j], acc.at[idxs[j]], ssems[j])

    def zero_acc():
        _zero2d(bufs[0], CHUNK, F1)

        def zfire(k, _):
            pltpu.make_async_copy(
                bufs[0], acc.at[pl.ds(tbase + k * CHUNK, CHUNK)], gsems[0]).start()
            return 0
        lax.fori_loop(0, nck, zfire, 0)

        def zdrain(k, _):
            pltpu.make_async_copy(
                bufs[0], acc.at[pl.ds(tbase + k * CHUNK, CHUNK)], gsems[0]).wait()
            return 0
        lax.fori_loop(0, nck, zdrain, 0)

    def stream(ngrp, rem, gath):
        def grp(g, _):
            for j in range(G):
                @pl.when(g > 0)
                def _(j=j):
                    scat(j).wait()
                for d in gath(g * G + j, j):
                    d.start()
            for j in range(G):
                for d in gath(g * G + j, j):
                    d.wait()
                scat(j).start(add=True)
            return 0
        lax.fori_loop(0, ngrp, grp, 0)
        if ngrp > 0:
            for j in range(G):
                scat(j).wait()
        for r in range(rem):
            for d in gath(ngrp * G + r, r):
                d.start()
                d.wait()
            pltpu.sync_copy(bufs[r], acc.at[idxs[r]], add=True)

    zero_acc()
    plsc.subcore_barrier()

    def gath_a(c, j):
        base = sid * EDGES_PER_TILE + c * CHUNK
        return (pltpu.make_async_copy(idxh.at[pl.ds(base, CHUNK)], idxs[j], gsems[j]),
                pltpu.make_async_copy(e1h.at[pl.ds(base, CHUNK), pl.ds(fb1, F1)],
                                      bufs[j], gsems[j]))
    stream(NG_A, REM_A, gath_a)
    plsc.subcore_barrier()

    def dump(dst):
        def dfire(k, _):
            r = tbase + k * CHUNK
            pltpu.make_async_copy(acc.at[pl.ds(r, CHUNK)], dst(r), gsems[0]).start()
            return 0
        lax.fori_loop(0, nck, dfire, 0)

        def ddrain(k, _):
            r = tbase + k * CHUNK
            pltpu.make_async_copy(acc.at[pl.ds(r, CHUNK)], dst(r), gsems[0]).wait()
            return 0
        lax.fori_loop(0, nck, ddrain, 0)

    dump(lambda r: s1o.at[pl.ds(r, CHUNK), pl.ds(fb1, F1)])

    zero_acc()
    plsc.subcore_barrier()

    def gath_b(c, j):
        base = (cid * NSUB + sid) * EDGES_PER_TILE_B + c * CHUNK
        return (pltpu.make_async_copy(idxh.at[pl.ds(base, CHUNK)], idxs[j], gsems[j]),
                pltpu.make_async_copy(e2h.at[pl.ds(base, CHUNK)], bufs[j], gsems[j]))
    stream(NG_B, REM_B, gath_b)
    plsc.subcore_barrier()

    dump(lambda r: s2o.at[pl.ds(cid * N + r, CHUNK)])


def _cnt_body(ridxh, sidxh, cro, cso, cacc,
              i0, i1, i2, i3, g0, g1, g2, g3, s0, s1, s2, s3, oneb):
    cid = lax.axis_index("c")
    sid = lax.axis_index("s")
    idxs = (i0, i1, i2, i3)
    gsems = (g0, g1, g2, g3)
    ssems = (s0, s1, s2, s3)

    _zero2d(oneb, CCHUNK, 128)
    tbase = sid * ROWS_MAIN
    nck = jnp.where(sid == NSUB - 1, ROWS_LAST // CCHUNK, ROWS_MAIN // CCHUNK)

    def zfire(k, _):
        pltpu.make_async_copy(
            oneb, cacc.at[pl.ds(tbase + k * CCHUNK, CCHUNK)], gsems[0]).start()
        return 0
    lax.fori_loop(0, nck, zfire, 0)

    def zdrain(k, _):
        pltpu.make_async_copy(
            oneb, cacc.at[pl.ds(tbase + k * CCHUNK, CCHUNK)], gsems[0]).wait()
        return 0
    lax.fori_loop(0, nck, zdrain, 0)

    ov = jnp.ones((16,), jnp.float32)

    def onerow(r, _):
        oneb[r, pl.ds(0, 16)] = ov
        return 0
    lax.fori_loop(0, CCHUNK, onerow, 0)
    plsc.subcore_barrier()

    def make_stream(idxh):
        def gath(c, j):
            base = sid * EDGES_PER_TILE + c * CCHUNK
            return pltpu.make_async_copy(idxh.at[pl.ds(base, CCHUNK)],
                                         idxs[j], gsems[j])

        def scat(j):
            return pltpu.make_async_copy(oneb, cacc.at[idxs[j]], ssems[j])

        def grp(g, _):
            for j in range(G):
                @pl.when(g > 0)
                def _(j=j):
                    scat(j).wait()
                gath(g * G + j, j).start()
            for j in range(G):
                gath(g * G + j, j).wait()
                scat(j).start(add=True)
            return 0
        lax.fori_loop(0, NG_A, grp, 0)
        for j in range(G):
            scat(j).wait()
        for r in range(REM_A):
            d = gath(NG_A * G + r, r)
            d.start()
            d.wait()
            pltpu.sync_copy(oneb, cacc.at[idxs[r]], add=True)

    @pl.when(cid == 0)
    def _():
        make_stream(ridxh)

    @pl.when(cid == 1)
    def _():
        make_stream(sidxh)

    plsc.subcore_barrier()

    def make_drow(out):
        def drow(k, _):
            r = tbase + k * CCHUNK
            pltpu.sync_copy(cacc.at[pl.ds(r, CCHUNK)], out.at[pl.ds(r, CCHUNK)])
            return 0
        return drow

    @pl.when(cid == 0)
    def _():
        lax.fori_loop(0, nck, make_drow(cro), 0)

    @pl.when(cid == 1)
    def _():
        lax.fori_loop(0, nck, make_drow(cso), 0)


@functools.cache
def _seg_scatter_kernel():
    return pl.kernel(
        _seg_body,
        out_type=[
            jax.ShapeDtypeStruct((N, D1), jnp.float32),
            jax.ShapeDtypeStruct((NCORES * N, D2), jnp.float32),
        ],
        mesh=plsc.VectorSubcoreMesh(core_axis_name="c", subcore_axis_name="s",
                                    num_cores=NCORES, num_subcores=NSUB),
        scratch_types=(
            [pltpu.VMEM_SHARED((N, F1), jnp.float32)]
            + [pltpu.VMEM((CHUNK, F1), jnp.float32)] * G
            + [pltpu.VMEM((CHUNK,), jnp.int32)] * G
            + [pltpu.SemaphoreType.DMA] * (2 * G)
        ),
    )


@functools.cache
def _cnt_kernel():
    return pl.kernel(
        _cnt_body,
        out_type=[
            jax.ShapeDtypeStruct((N, 128), jnp.float32),
            jax.ShapeDtypeStruct((N, 128), jnp.float32),
        ],
        mesh=plsc.VectorSubcoreMesh(core_axis_name="c", subcore_axis_name="s",
                                    num_cores=NCORES, num_subcores=NSUB),
        scratch_types=(
            [pltpu.VMEM_SHARED((N, 128), jnp.float32)]
            + [pltpu.VMEM((CCHUNK,), jnp.int32)] * G
            + [pltpu.SemaphoreType.DMA] * (2 * G)
            + [pltpu.VMEM((CCHUNK, 128), jnp.float32)]
        ),
    )


def _seg_scatter(e1, e2, idx):
    return _seg_scatter_kernel()(e1, e2, idx)


def _node_mlp_body(nf, i1, o1, i2a, i2b, o2a, o2b, cr, cs, se1, se2, gf,
                   Wn1, Win1, Wout1, bn1, Wn2, Win2, Wout2, bn2,
                   Wg1, Wng1, Weg1, bg1, Wg2, Wng2, Weg2, bg2,
                   Wm, bm, Wls, bls, mo, lo, sn1, sn2):
    i = pl.program_id(0)

    @pl.when(i == 0)
    def _():
        sn1[...] = jnp.zeros_like(sn1)
        sn2[...] = jnp.zeros_like(sn2)

    crv = jnp.maximum(cr[:, 0:1], 1.0)
    csv = jnp.maximum(cs[:, 0:1], 1.0)
    dot = _bdot
    n1 = jnp.maximum(
        dot(nf[...], Wn1[...]) + dot(i1[...] / crv, Win1[...])
        + dot(o1[...] / csv, Wout1[...]) + bn1[...], 0.0)
    sn1[...] += jnp.sum(n1, axis=0, keepdims=True)
    n2 = jnp.maximum(
        dot(n1, Wn2[...]) + dot((i2a[...] + i2b[...]) / crv, Win2[...])
        + dot((o2a[...] + o2b[...]) / csv, Wout2[...]) + bn2[...], 0.0)
    sn2[...] += jnp.sum(n2, axis=0, keepdims=True)

    @pl.when(i == pl.num_programs(0) - 1)
    def _():
        g1 = jnp.maximum(
            dot(gf[...], Wg1[...]) + dot(sn1[...] / N, Wng1[...])
            + dot(se1[...] / E, Weg1[...]) + bg1[...], 0.0)
        g2 = jnp.maximum(
            dot(g1, Wg2[...]) + dot(sn2[...] / N, Wng2[...])
            + dot(se2[...] / E, Weg2[...]) + bg2[...], 0.0)
        mo[...] = dot(g2, Wm[...]) + bm[...]
        lo[...] = jnp.clip(dot(g2, Wls[...]) + bls[...],
                           LOG_SIG_MIN, LOG_SIG_MAX)


def _node_mlp(nf, i1, o1, i2a, i2b, o2a, o2b, cr, cs, se1, se2, gf, w):
    grid = N // NB
    full = lambda a, b: pl.BlockSpec((a, b), lambda i: (0, 0))
    return pl.pallas_call(
        _node_mlp_body,
        grid=(grid,),
        in_specs=[
            pl.BlockSpec((NB, D_NODE), lambda i: (i, 0)),
            pl.BlockSpec((NB, D1), lambda i: (i, 0)),
            pl.BlockSpec((NB, D1), lambda i: (i, 0)),
            pl.BlockSpec((NB, D2), lambda i: (i, 0)),
            pl.BlockSpec((NB, D2), lambda i: (i, 0)),
            pl.BlockSpec((NB, D2), lambda i: (i, 0)),
            pl.BlockSpec((NB, D2), lambda i: (i, 0)),
            pl.BlockSpec((NB, 128), lambda i: (i, 0)),
            pl.BlockSpec((NB, 128), lambda i: (i, 0)),
            full(1, D1), full(1, D2), full(1, D_GLOBAL := 16),
            full(D_NODE, D1), full(D1, D1), full(D1, D1), full(1, D1),
            full(D1, D2), full(D2, D2), full(D2, D2), full(1, D2),
            full(16, D1), full(D1, D1), full(D1, D1), full(1, D1),
            full(D1, D2), full(D2, D2), full(D2, D2), full(1, D2),
            full(D2, A), full(1, A), full(D2, A), full(1, A),
        ],
        out_specs=[
            pl.BlockSpec((1, A), lambda i: (0, 0)),
            pl.BlockSpec((1, A), lambda i: (0, 0)),
        ],
        out_shape=[
            jax.ShapeDtypeStruct((1, A), jnp.float32),
            jax.ShapeDtypeStruct((1, A), jnp.float32),
        ],
        scratch_shapes=[
            pltpu.VMEM((1, D1), jnp.float32),
            pltpu.VMEM((1, D2), jnp.float32),
        ],
        compiler_params=pltpu.CompilerParams(
            dimension_semantics=("arbitrary",)),
    )(nf, i1, o1, i2a, i2b, o2a, o2b, cr, cs, se1, se2, gf,
      w['Wn1'], w['Win1'], w['Wout1'], w['bn1'].reshape(1, D1),
      w['Wn2'], w['Win2'], w['Wout2'], w['bn2'].reshape(1, D2),
      w['Wg1'], w['Wng1'], w['Weg1'], w['bg1'].reshape(1, D1),
      w['Wg2'], w['Wng2'], w['Weg2'], w['bg2'].reshape(1, D2),
      w['Wm'], w['bm'].reshape(1, A), w['Wls'], w['bls'].reshape(1, A))


def kernel(node_features, edge_features, global_features, senders, receivers, params):
    p = params
    e1, e2, se1, se2 = _edge_mlp(
        edge_features, p['We1'], p['be1'].reshape(1, D1),
        p['We2'], p['be2'].reshape(1, D2))
    i1, i2p = _seg_scatter(e1, e2, receivers)
    o1, o2p = _seg_scatter(e1, e2, senders)
    cr, cs = _cnt_kernel()(receivers, senders)
    mean, log_std = _node_mlp(node_features, i1, o1,
                              i2p[:N], i2p[N:], o2p[:N], o2p[N:], cr, cs,
                              se1, se2, global_features, params)
    return mean, log_std


# speedup vs baseline: 1.0065x; 1.0065x over previous
import functools

import jax
import jax.numpy as jnp
from jax import lax
from jax.experimental import pallas as pl
from jax.experimental.pallas import tpu as pltpu
from jax.experimental.pallas import tpu_sc as plsc

N = 10000
E = 320000
D_NODE = 128
D_EDGE = 16
D1 = 256
D2 = 128
A = 8
LOG_SIG_MAX = 2.0
LOG_SIG_MIN = -20.0

NCORES = 2
NSUB = 16
F1 = D1 // NCORES
CHUNK = 80
EDGES_PER_TILE = E // NSUB
EDGES_PER_TILE_B = E // (NCORES * NSUB)
ROWS_MAIN = 640
ROWS_LAST = N - (NSUB - 1) * ROWS_MAIN
CCHUNK = 80

EB = 512
NB = 1000


def _bdot(a, b):
    return jnp.dot(a.astype(jnp.bfloat16), b.astype(jnp.bfloat16),
                   preferred_element_type=jnp.float32)


def _edge_mlp_body(ef, We1, be1, We2, be2, e1o, e2o, se1o, se2o, acc1, acc2):
    i = pl.program_id(0)

    @pl.when(i == 0)
    def _():
        acc1[...] = jnp.zeros_like(acc1)
        acc2[...] = jnp.zeros_like(acc2)

    h = jnp.maximum(
        _bdot(ef[...], We1[...]) + be1[...], 0.0)
    e1o[...] = h
    z = jnp.maximum(
        _bdot(h, We2[...]) + be2[...], 0.0)
    e2o[...] = z
    acc1[...] += jnp.sum(h, axis=0, keepdims=True)
    acc2[...] += jnp.sum(z, axis=0, keepdims=True)

    @pl.when(i == pl.num_programs(0) - 1)
    def _():
        se1o[...] = acc1[...]
        se2o[...] = acc2[...]


def _edge_mlp(ef, We1, be1, We2, be2):
    grid = E // EB
    return pl.pallas_call(
        _edge_mlp_body,
        grid=(grid,),
        in_specs=[
            pl.BlockSpec((EB, D_EDGE), lambda i: (i, 0)),
            pl.BlockSpec((D_EDGE, D1), lambda i: (0, 0)),
            pl.BlockSpec((1, D1), lambda i: (0, 0)),
            pl.BlockSpec((D1, D2), lambda i: (0, 0)),
            pl.BlockSpec((1, D2), lambda i: (0, 0)),
        ],
        out_specs=[
            pl.BlockSpec((EB, D1), lambda i: (i, 0)),
            pl.BlockSpec((EB, D2), lambda i: (i, 0)),
            pl.BlockSpec((1, D1), lambda i: (0, 0)),
            pl.BlockSpec((1, D2), lambda i: (0, 0)),
        ],
        out_shape=[
            jax.ShapeDtypeStruct((E, D1), jnp.float32),
            jax.ShapeDtypeStruct((E, D2), jnp.float32),
            jax.ShapeDtypeStruct((1, D1), jnp.float32),
            jax.ShapeDtypeStruct((1, D2), jnp.float32),
        ],
        scratch_shapes=[
            pltpu.VMEM((1, D1), jnp.float32),
            pltpu.VMEM((1, D2), jnp.float32),
        ],
        compiler_params=pltpu.CompilerParams(
            dimension_semantics=("arbitrary",)),
    )(ef, We1, be1, We2, be2)


def _zero2d(ref, rows, width):
    zv = jnp.zeros((16,), jnp.float32)

    def row(r, _):
        def col(cc, _):
            ref[r, pl.ds(cc * 16, 16)] = zv
            return 0
        lax.fori_loop(0, width // 16, col, 0)
        return 0
    lax.fori_loop(0, rows, row, 0)


G = 4
NG_A = (E // NSUB) // CHUNK // G
REM_A = (E // NSUB) // CHUNK - NG_A * G
NG_B = EDGES_PER_TILE_B // CHUNK // G
REM_B = EDGES_PER_TILE_B // CHUNK - NG_B * G


def _sc_body(e1h, e2h, ridxh, sidxh, s1r, s2r, s1s, s2s, cro, cso, acc,
             b0, b1, b2, b3, i0, i1, i2, i3,
             g0, g1, g2, g3, s0, s1, s2, s3):
    cid = lax.axis_index("c")
    sid = lax.axis_index("s")
    fb1 = cid * F1
    tbase = sid * ROWS_MAIN
    nck = jnp.where(sid == NSUB - 1, ROWS_LAST // CHUNK, ROWS_MAIN // CHUNK)
    bufs = (b0, b1, b2, b3)
    idxs = (i0, i1, i2, i3)
    gsems = (g0, g1, g2, g3)
    ssems = (s0, s1, s2, s3)

    def zero_acc():
        _zero2d(bufs[0], CHUNK, F1)

        def zfire(k, _):
            pltpu.make_async_copy(
                bufs[0], acc.at[pl.ds(tbase + k * CHUNK, CHUNK)], gsems[0]).start()
            return 0
        lax.fori_loop(0, nck, zfire, 0)

        def zdrain(k, _):
            pltpu.make_async_copy(
                bufs[0], acc.at[pl.ds(tbase + k * CHUNK, CHUNK)], gsems[0]).wait()
            return 0
        lax.fori_loop(0, nck, zdrain, 0)

    def stream(ngrp, rem, gath, src=None):
        def scat(j):
            return pltpu.make_async_copy(
                bufs[j] if src is None else src, acc.at[idxs[j]], ssems[j])

        def grp(g, _):
            for j in range(G):
                @pl.when(g > 0)
                def _(j=j):
                    scat(j).wait()
                for d in gath(g * G + j, j):
                    d.start()
            for j in range(G):
                for d in gath(g * G + j, j):
                    d.wait()
                scat(j).start(add=True)
            return 0
        lax.fori_loop(0, ngrp, grp, 0)
        if ngrp > 0:
            for j in range(G):
                scat(j).wait()
        for r in range(rem):
            for d in gath(ngrp * G + r, r):
                d.start()
                d.wait()
            pltpu.sync_copy(bufs[r] if src is None else src,
                            acc.at[idxs[r]], add=True)

    def dump(dst):
        def dfire(k, _):
            r = tbase + k * CHUNK
            pltpu.make_async_copy(acc.at[pl.ds(r, CHUNK)], dst(r), gsems[0]).start()
            return 0
        lax.fori_loop(0, nck, dfire, 0)

        def ddrain(k, _):
            r = tbase + k * CHUNK
            pltpu.make_async_copy(acc.at[pl.ds(r, CHUNK)], dst(r), gsems[0]).wait()
            return 0
        lax.fori_loop(0, nck, ddrain, 0)

    def gath_a(idxh):
        def gath(c, j):
            base = sid * EDGES_PER_TILE + c * CHUNK
            return (pltpu.make_async_copy(idxh.at[pl.ds(base, CHUNK)],
                                          idxs[j], gsems[j]),
                    pltpu.make_async_copy(
                        e1h.at[pl.ds(base, CHUNK), pl.ds(fb1, F1)],
                        bufs[j], gsems[j]))
        return gath

    def gath_b(idxh):
        def gath(c, j):
            base = (cid * NSUB + sid) * EDGES_PER_TILE_B + c * CHUNK
            return (pltpu.make_async_copy(idxh.at[pl.ds(base, CHUNK)],
                                          idxs[j], gsems[j]),
                    pltpu.make_async_copy(e2h.at[pl.ds(base, CHUNK)],
                                          bufs[j], gsems[j]))
        return gath

    for idxh, s1o, s2o in ((ridxh, s1r, s2r), (sidxh, s1s, s2s)):
        zero_acc()
        plsc.subcore_barrier()
        stream(NG_A, REM_A, gath_a(idxh))
        plsc.subcore_barrier()
        dump(lambda r, s1o=s1o: s1o.at[pl.ds(r, CHUNK), pl.ds(fb1, F1)])

        zero_acc()
        plsc.subcore_barrier()
        stream(NG_B, REM_B, gath_b(idxh))
        plsc.subcore_barrier()
        dump(lambda r, s2o=s2o: s2o.at[pl.ds(cid * N + r, CHUNK)])

    zero_acc()
    ov = jnp.ones((16,), jnp.float32)

    def onerow(r, _):
        bufs[0][r, pl.ds(0, 16)] = ov
        return 0
    lax.fori_loop(0, CHUNK, onerow, 0)
    plsc.subcore_barrier()

    def gath_c(idxh):
        def gath(c, j):
            base = sid * EDGES_PER_TILE + c * CHUNK
            return (pltpu.make_async_copy(idxh.at[pl.ds(base, CHUNK)],
                                          idxs[j], gsems[j]),)
        return gath

    @pl.when(cid == 0)
    def _():
        stream(NG_A, REM_A, gath_c(ridxh), src=bufs[0])

    @pl.when(cid == 1)
    def _():
        stream(NG_A, REM_A, gath_c(sidxh), src=bufs[0])

    plsc.subcore_barrier()

    @pl.when(cid == 0)
    def _():
        dump(lambda r: cro.at[pl.ds(r, CHUNK)])

    @pl.when(cid == 1)
    def _():
        dump(lambda r: cso.at[pl.ds(r, CHUNK)])


@functools.cache
def _sc_kernel():
    return pl.kernel(
        _sc_body,
        out_type=[
            jax.ShapeDtypeStruct((N, D1), jnp.float32),
            jax.ShapeDtypeStruct((NCORES * N, D2), jnp.float32),
            jax.ShapeDtypeStruct((N, D1), jnp.float32),
            jax.ShapeDtypeStruct((NCORES * N, D2), jnp.float32),
            jax.ShapeDtypeStruct((N, 128), jnp.float32),
            jax.ShapeDtypeStruct((N, 128), jnp.float32),
        ],
        mesh=plsc.VectorSubcoreMesh(core_axis_name="c", subcore_axis_name="s",
                                    num_cores=NCORES, num_subcores=NSUB),
        scratch_types=(
            [pltpu.VMEM_SHARED((N, F1), jnp.float32)]
            + [pltpu.VMEM((CHUNK, F1), jnp.float32)] * G
            + [pltpu.VMEM((CHUNK,), jnp.int32)] * G
            + [pltpu.SemaphoreType.DMA] * (2 * G)
        ),
    )


def _node_mlp_body(nf, i1, o1, i2a, i2b, o2a, o2b, cr, cs, se1, se2, gf,
                   Wn1, Win1, Wout1, bn1, Wn2, Win2, Wout2, bn2,
                   Wg1, Wng1, Weg1, bg1, Wg2, Wng2, Weg2, bg2,
                   Wm, bm, Wls, bls, mo, lo, sn1, sn2):
    i = pl.program_id(0)

    @pl.when(i == 0)
    def _():
        sn1[...] = jnp.zeros_like(sn1)
        sn2[...] = jnp.zeros_like(sn2)

    crv = jnp.maximum(cr[:, 0:1], 1.0)
    csv = jnp.maximum(cs[:, 0:1], 1.0)
    dot = _bdot
    n1 = jnp.maximum(
        dot(nf[...], Wn1[...]) + dot(i1[...] / crv, Win1[...])
        + dot(o1[...] / csv, Wout1[...]) + bn1[...], 0.0)
    sn1[...] += jnp.sum(n1, axis=0, keepdims=True)
    n2 = jnp.maximum(
        dot(n1, Wn2[...]) + dot((i2a[...] + i2b[...]) / crv, Win2[...])
        + dot((o2a[...] + o2b[...]) / csv, Wout2[...]) + bn2[...], 0.0)
    sn2[...] += jnp.sum(n2, axis=0, keepdims=True)

    @pl.when(i == pl.num_programs(0) - 1)
    def _():
        g1 = jnp.maximum(
            dot(gf[...], Wg1[...]) + dot(sn1[...] / N, Wng1[...])
            + dot(se1[...] / E, Weg1[...]) + bg1[...], 0.0)
        g2 = jnp.maximum(
            dot(g1, Wg2[...]) + dot(sn2[...] / N, Wng2[...])
            + dot(se2[...] / E, Weg2[...]) + bg2[...], 0.0)
        mo[...] = dot(g2, Wm[...]) + bm[...]
        lo[...] = jnp.clip(dot(g2, Wls[...]) + bls[...],
                           LOG_SIG_MIN, LOG_SIG_MAX)


def _node_mlp(nf, i1, o1, i2a, i2b, o2a, o2b, cr, cs, se1, se2, gf, w):
    grid = N // NB
    full = lambda a, b: pl.BlockSpec((a, b), lambda i: (0, 0))
    return pl.pallas_call(
        _node_mlp_body,
        grid=(grid,),
        in_specs=[
            pl.BlockSpec((NB, D_NODE), lambda i: (i, 0)),
            pl.BlockSpec((NB, D1), lambda i: (i, 0)),
            pl.BlockSpec((NB, D1), lambda i: (i, 0)),
            pl.BlockSpec((NB, D2), lambda i: (i, 0)),
            pl.BlockSpec((NB, D2), lambda i: (i + N // NB, 0)),
            pl.BlockSpec((NB, D2), lambda i: (i, 0)),
            pl.BlockSpec((NB, D2), lambda i: (i + N // NB, 0)),
            pl.BlockSpec((NB, 128), lambda i: (i, 0)),
            pl.BlockSpec((NB, 128), lambda i: (i, 0)),
            full(1, D1), full(1, D2), full(1, D_GLOBAL := 16),
            full(D_NODE, D1), full(D1, D1), full(D1, D1), full(1, D1),
            full(D1, D2), full(D2, D2), full(D2, D2), full(1, D2),
            full(16, D1), full(D1, D1), full(D1, D1), full(1, D1),
            full(D1, D2), full(D2, D2), full(D2, D2), full(1, D2),
            full(D2, A), full(1, A), full(D2, A), full(1, A),
        ],
        out_specs=[
            pl.BlockSpec((1, A), lambda i: (0, 0)),
            pl.BlockSpec((1, A), lambda i: (0, 0)),
        ],
        out_shape=[
            jax.ShapeDtypeStruct((1, A), jnp.float32),
            jax.ShapeDtypeStruct((1, A), jnp.float32),
        ],
        scratch_shapes=[
            pltpu.VMEM((1, D1), jnp.float32),
            pltpu.VMEM((1, D2), jnp.float32),
        ],
        compiler_params=pltpu.CompilerParams(
            dimension_semantics=("arbitrary",)),
    )(nf, i1, o1, i2a, i2b, o2a, o2b, cr, cs, se1, se2, gf,
      w['Wn1'], w['Win1'], w['Wout1'], w['bn1'].reshape(1, D1),
      w['Wn2'], w['Win2'], w['Wout2'], w['bn2'].reshape(1, D2),
      w['Wg1'], w['Wng1'], w['Weg1'], w['bg1'].reshape(1, D1),
      w['Wg2'], w['Wng2'], w['Weg2'], w['bg2'].reshape(1, D2),
      w['Wm'], w['bm'].reshape(1, A), w['Wls'], w['bls'].reshape(1, A))


def _sc_scatter(e1, e2, receivers, senders):
    return _sc_kernel()(e1, e2, receivers, senders)


def kernel(node_features, edge_features, global_features, senders, receivers, params):
    p = params
    e1, e2, se1, se2 = _edge_mlp(
        edge_features, p['We1'], p['be1'].reshape(1, D1),
        p['We2'], p['be2'].reshape(1, D2))
    i1, i2p, o1, o2p, cr, cs = _sc_scatter(e1, e2, receivers, senders)
    mean, log_std = _node_mlp(node_features, i1, o1,
                              i2p, i2p, o2p, o2p, cr, cs,
                              se1, se2, global_features, params)
    return mean, log_std


# speedup vs baseline: 1.3225x; 1.3139x over previous
import functools

import jax
import jax.numpy as jnp
from jax import lax
from jax.experimental import pallas as pl
from jax.experimental.pallas import tpu as pltpu
from jax.experimental.pallas import tpu_sc as plsc

N = 10000
E = 320000
D_NODE = 128
D_EDGE = 16
D1 = 256
D2 = 128
A = 8
LOG_SIG_MAX = 2.0
LOG_SIG_MIN = -20.0

NCORES = 2
NSUB = 16
F1 = D1 // NCORES
CHUNK = 80
EDGES_PER_TILE = E // NSUB
EDGES_PER_TILE_B = E // (NCORES * NSUB)
ROWS_MAIN = 640
ROWS_LAST = N - (NSUB - 1) * ROWS_MAIN
CCHUNK = 80

EB = 8000
NB = 2000


def _bdot(a, b):
    return jnp.dot(a.astype(jnp.bfloat16), b.astype(jnp.bfloat16),
                   preferred_element_type=jnp.float32)


def _edge_mlp_body(ef, We1, be1, We2, be2, e1o, e2o, se1o, se2o, acc1, acc2):
    i = pl.program_id(0)

    @pl.when(i == 0)
    def _():
        acc1[...] = jnp.zeros_like(acc1)
        acc2[...] = jnp.zeros_like(acc2)

    h = jnp.maximum(
        _bdot(ef[...], We1[...]) + be1[...], 0.0)
    e1o[...] = h
    z = jnp.maximum(
        _bdot(h, We2[...]) + be2[...], 0.0)
    e2o[...] = z
    acc1[...] += jnp.sum(h, axis=0, keepdims=True)
    acc2[...] += jnp.sum(z, axis=0, keepdims=True)

    @pl.when(i == pl.num_programs(0) - 1)
    def _():
        se1o[...] = acc1[...]
        se2o[...] = acc2[...]


def _edge_mlp(ef, We1, be1, We2, be2):
    grid = E // EB
    return pl.pallas_call(
        _edge_mlp_body,
        grid=(grid,),
        in_specs=[
            pl.BlockSpec((EB, D_EDGE), lambda i: (i, 0)),
            pl.BlockSpec((D_EDGE, D1), lambda i: (0, 0)),
            pl.BlockSpec((1, D1), lambda i: (0, 0)),
            pl.BlockSpec((D1, D2), lambda i: (0, 0)),
            pl.BlockSpec((1, D2), lambda i: (0, 0)),
        ],
        out_specs=[
            pl.BlockSpec((EB, D1), lambda i: (i, 0)),
            pl.BlockSpec((EB, D2), lambda i: (i, 0)),
            pl.BlockSpec((1, D1), lambda i: (0, 0)),
            pl.BlockSpec((1, D2), lambda i: (0, 0)),
        ],
        out_shape=[
            jax.ShapeDtypeStruct((E, D1), jnp.float32),
            jax.ShapeDtypeStruct((E, D2), jnp.float32),
            jax.ShapeDtypeStruct((1, D1), jnp.float32),
            jax.ShapeDtypeStruct((1, D2), jnp.float32),
        ],
        scratch_shapes=[
            pltpu.VMEM((1, D1), jnp.float32),
            pltpu.VMEM((1, D2), jnp.float32),
        ],
        compiler_params=pltpu.CompilerParams(
            dimension_semantics=("arbitrary",)),
    )(ef, We1, be1, We2, be2)


def _zero2d(ref, rows, width):
    zv = jnp.zeros((16,), jnp.float32)

    def row(r, _):
        def col(cc, _):
            ref[r, pl.ds(cc * 16, 16)] = zv
            return 0
        lax.fori_loop(0, width // 16, col, 0)
        return 0
    lax.fori_loop(0, rows, row, 0)


G = 4
NG_A = (E // NSUB) // CHUNK // G
REM_A = (E // NSUB) // CHUNK - NG_A * G
NG_B = EDGES_PER_TILE_B // CHUNK // G
REM_B = EDGES_PER_TILE_B // CHUNK - NG_B * G


def _sc_body(e1h, e2h, ridxh, sidxh, s1r, s2r, s1s, s2s, cro, cso, acc,
             b0, b1, b2, b3, i0, i1, i2, i3,
             g0, g1, g2, g3, s0, s1, s2, s3):
    cid = lax.axis_index("c")
    sid = lax.axis_index("s")
    fb1 = cid * F1
    tbase = sid * ROWS_MAIN
    nck = jnp.where(sid == NSUB - 1, ROWS_LAST // CHUNK, ROWS_MAIN // CHUNK)
    bufs = (b0, b1, b2, b3)
    idxs = (i0, i1, i2, i3)
    gsems = (g0, g1, g2, g3)
    ssems = (s0, s1, s2, s3)

    def zero_acc():
        _zero2d(bufs[0], CHUNK, F1)

        def zfire(k, _):
            pltpu.make_async_copy(
                bufs[0], acc.at[pl.ds(tbase + k * CHUNK, CHUNK)], gsems[0]).start()
            return 0
        lax.fori_loop(0, nck, zfire, 0)

        def zdrain(k, _):
            pltpu.make_async_copy(
                bufs[0], acc.at[pl.ds(tbase + k * CHUNK, CHUNK)], gsems[0]).wait()
            return 0
        lax.fori_loop(0, nck, zdrain, 0)

    def stream(ngrp, rem, gath, src=None):
        def scat(j):
            return pltpu.make_async_copy(
                bufs[j] if src is None else src, acc.at[idxs[j]], ssems[j])

        def grp(g, _):
            for j in range(G):
                @pl.when(g > 0)
                def _(j=j):
                    scat(j).wait()
                for d in gath(g * G + j, j):
                    d.start()
            for j in range(G):
                for d in gath(g * G + j, j):
                    d.wait()
                scat(j).start(add=True)
            return 0
        lax.fori_loop(0, ngrp, grp, 0)
        if ngrp > 0:
            for j in range(G):
                scat(j).wait()
        for r in range(rem):
            for d in gath(ngrp * G + r, r):
                d.start()
                d.wait()
            pltpu.sync_copy(bufs[r] if src is None else src,
                            acc.at[idxs[r]], add=True)

    def dump(dst):
        def dfire(k, _):
            r = tbase + k * CHUNK
            pltpu.make_async_copy(acc.at[pl.ds(r, CHUNK)], dst(r), gsems[0]).start()
            return 0
        lax.fori_loop(0, nck, dfire, 0)

        def ddrain(k, _):
            r = tbase + k * CHUNK
            pltpu.make_async_copy(acc.at[pl.ds(r, CHUNK)], dst(r), gsems[0]).wait()
            return 0
        lax.fori_loop(0, nck, ddrain, 0)

    def gath_a(idxh):
        def gath(c, j):
            base = sid * EDGES_PER_TILE + c * CHUNK
            return (pltpu.make_async_copy(idxh.at[pl.ds(base, CHUNK)],
                                          idxs[j], gsems[j]),
                    pltpu.make_async_copy(
                        e1h.at[pl.ds(base, CHUNK), pl.ds(fb1, F1)],
                        bufs[j], gsems[j]))
        return gath

    def gath_b(idxh):
        def gath(c, j):
            base = (cid * NSUB + sid) * EDGES_PER_TILE_B + c * CHUNK
            return (pltpu.make_async_copy(idxh.at[pl.ds(base, CHUNK)],
                                          idxs[j], gsems[j]),
                    pltpu.make_async_copy(e2h.at[pl.ds(base, CHUNK)],
                                          bufs[j], gsems[j]))
        return gath

    for idxh, s1o, s2o in ((ridxh, s1r, s2r), (sidxh, s1s, s2s)):
        zero_acc()
        plsc.subcore_barrier()
        stream(NG_A, REM_A, gath_a(idxh))
        plsc.subcore_barrier()
        dump(lambda r, s1o=s1o: s1o.at[pl.ds(r, CHUNK), pl.ds(fb1, F1)])

        zero_acc()
        plsc.subcore_barrier()
        stream(NG_B, REM_B, gath_b(idxh))
        plsc.subcore_barrier()
        dump(lambda r, s2o=s2o: s2o.at[pl.ds(cid * N + r, CHUNK)])

    zero_acc()
    ov = jnp.ones((16,), jnp.float32)

    def onerow(r, _):
        bufs[0][r, pl.ds(0, 16)] = ov
        return 0
    lax.fori_loop(0, CHUNK, onerow, 0)
    plsc.subcore_barrier()

    def gath_c(idxh):
        def gath(c, j):
            base = sid * EDGES_PER_TILE + c * CHUNK
            return (pltpu.make_async_copy(idxh.at[pl.ds(base, CHUNK)],
                                          idxs[j], gsems[j]),)
        return gath

    @pl.when(cid == 0)
    def _():
        stream(NG_A, REM_A, gath_c(ridxh), src=bufs[0])

    @pl.when(cid == 1)
    def _():
        stream(NG_A, REM_A, gath_c(sidxh), src=bufs[0])

    plsc.subcore_barrier()

    @pl.when(cid == 0)
    def _():
        dump(lambda r: cro.at[pl.ds(r, CHUNK)])

    @pl.when(cid == 1)
    def _():
        dump(lambda r: cso.at[pl.ds(r, CHUNK)])


@functools.cache
def _sc_kernel():
    return pl.kernel(
        _sc_body,
        out_type=[
            jax.ShapeDtypeStruct((N, D1), jnp.float32),
            jax.ShapeDtypeStruct((NCORES * N, D2), jnp.float32),
            jax.ShapeDtypeStruct((N, D1), jnp.float32),
            jax.ShapeDtypeStruct((NCORES * N, D2), jnp.float32),
            jax.ShapeDtypeStruct((N, 128), jnp.float32),
            jax.ShapeDtypeStruct((N, 128), jnp.float32),
        ],
        mesh=plsc.VectorSubcoreMesh(core_axis_name="c", subcore_axis_name="s",
                                    num_cores=NCORES, num_subcores=NSUB),
        scratch_types=(
            [pltpu.VMEM_SHARED((N, F1), jnp.float32)]
            + [pltpu.VMEM((CHUNK, F1), jnp.float32)] * G
            + [pltpu.VMEM((CHUNK,), jnp.int32)] * G
            + [pltpu.SemaphoreType.DMA] * (2 * G)
        ),
    )


def _node_mlp_body(nf, i1, o1, i2a, i2b, o2a, o2b, cr, cs, se1, se2, gf,
                   Wn1, Win1, Wout1, bn1, Wn2, Win2, Wout2, bn2,
                   Wg1, Wng1, Weg1, bg1, Wg2, Wng2, Weg2, bg2,
                   Wm, bm, Wls, bls, mo, lo, sn1, sn2):
    i = pl.program_id(0)

    @pl.when(i == 0)
    def _():
        sn1[...] = jnp.zeros_like(sn1)
        sn2[...] = jnp.zeros_like(sn2)

    crv = jnp.maximum(cr[:, 0:1], 1.0)
    csv = jnp.maximum(cs[:, 0:1], 1.0)
    dot = _bdot
    n1 = jnp.maximum(
        dot(nf[...], Wn1[...]) + dot(i1[...] / crv, Win1[...])
        + dot(o1[...] / csv, Wout1[...]) + bn1[...], 0.0)
    sn1[...] += jnp.sum(n1, axis=0, keepdims=True)
    n2 = jnp.maximum(
        dot(n1, Wn2[...]) + dot((i2a[...] + i2b[...]) / crv, Win2[...])
        + dot((o2a[...] + o2b[...]) / csv, Wout2[...]) + bn2[...], 0.0)
    sn2[...] += jnp.sum(n2, axis=0, keepdims=True)

    @pl.when(i == pl.num_programs(0) - 1)
    def _():
        g1 = jnp.maximum(
            dot(gf[...], Wg1[...]) + dot(sn1[...] / N, Wng1[...])
            + dot(se1[...] / E, Weg1[...]) + bg1[...], 0.0)
        g2 = jnp.maximum(
            dot(g1, Wg2[...]) + dot(sn2[...] / N, Wng2[...])
            + dot(se2[...] / E, Weg2[...]) + bg2[...], 0.0)
        mo[...] = dot(g2, Wm[...]) + bm[...]
        lo[...] = jnp.clip(dot(g2, Wls[...]) + bls[...],
                           LOG_SIG_MIN, LOG_SIG_MAX)


def _node_mlp(nf, i1, o1, i2a, i2b, o2a, o2b, cr, cs, se1, se2, gf, w):
    grid = N // NB
    full = lambda a, b: pl.BlockSpec((a, b), lambda i: (0, 0))
    return pl.pallas_call(
        _node_mlp_body,
        grid=(grid,),
        in_specs=[
            pl.BlockSpec((NB, D_NODE), lambda i: (i, 0)),
            pl.BlockSpec((NB, D1), lambda i: (i, 0)),
            pl.BlockSpec((NB, D1), lambda i: (i, 0)),
            pl.BlockSpec((NB, D2), lambda i: (i, 0)),
            pl.BlockSpec((NB, D2), lambda i: (i + N // NB, 0)),
            pl.BlockSpec((NB, D2), lambda i: (i, 0)),
            pl.BlockSpec((NB, D2), lambda i: (i + N // NB, 0)),
            pl.BlockSpec((NB, 128), lambda i: (i, 0)),
            pl.BlockSpec((NB, 128), lambda i: (i, 0)),
            full(1, D1), full(1, D2), full(1, D_GLOBAL := 16),
            full(D_NODE, D1), full(D1, D1), full(D1, D1), full(1, D1),
            full(D1, D2), full(D2, D2), full(D2, D2), full(1, D2),
            full(16, D1), full(D1, D1), full(D1, D1), full(1, D1),
            full(D1, D2), full(D2, D2), full(D2, D2), full(1, D2),
            full(D2, A), full(1, A), full(D2, A), full(1, A),
        ],
        out_specs=[
            pl.BlockSpec((1, A), lambda i: (0, 0)),
            pl.BlockSpec((1, A), lambda i: (0, 0)),
        ],
        out_shape=[
            jax.ShapeDtypeStruct((1, A), jnp.float32),
            jax.ShapeDtypeStruct((1, A), jnp.float32),
        ],
        scratch_shapes=[
            pltpu.VMEM((1, D1), jnp.float32),
            pltpu.VMEM((1, D2), jnp.float32),
        ],
        compiler_params=pltpu.CompilerParams(
            dimension_semantics=("arbitrary",)),
    )(nf, i1, o1, i2a, i2b, o2a, o2b, cr, cs, se1, se2, gf,
      w['Wn1'], w['Win1'], w['Wout1'], w['bn1'].reshape(1, D1),
      w['Wn2'], w['Win2'], w['Wout2'], w['bn2'].reshape(1, D2),
      w['Wg1'], w['Wng1'], w['Weg1'], w['bg1'].reshape(1, D1),
      w['Wg2'], w['Wng2'], w['Weg2'], w['bg2'].reshape(1, D2),
      w['Wm'], w['bm'].reshape(1, A), w['Wls'], w['bls'].reshape(1, A))


def _sc_scatter(e1, e2, receivers, senders):
    return _sc_kernel()(e1, e2, receivers, senders)


def kernel(node_features, edge_features, global_features, senders, receivers, params):
    p = params
    e1, e2, se1, se2 = _edge_mlp(
        edge_features, p['We1'], p['be1'].reshape(1, D1),
        p['We2'], p['be2'].reshape(1, D2))
    i1, i2p, o1, o2p, cr, cs = _sc_scatter(e1, e2, receivers, senders)
    mean, log_std = _node_mlp(node_features, i1, o1,
                              i2p, i2p, o2p, o2p, cr, cs,
                              se1, se2, global_features, params)
    return mean, log_std


# speedup vs baseline: 1.3257x; 1.0024x over previous
import functools

import jax
import jax.numpy as jnp
from jax import lax
from jax.experimental import pallas as pl
from jax.experimental.pallas import tpu as pltpu
from jax.experimental.pallas import tpu_sc as plsc

N = 10000
E = 320000
D_NODE = 128
D_EDGE = 16
D1 = 256
D2 = 128
A = 8
LOG_SIG_MAX = 2.0
LOG_SIG_MIN = -20.0

NCORES = 2
NSUB = 16
F1 = D1 // NCORES
CHUNK = 80
EDGES_PER_TILE = E // NSUB
EDGES_PER_TILE_B = E // (NCORES * NSUB)
ROWS_MAIN = 640
ROWS_LAST = N - (NSUB - 1) * ROWS_MAIN
CCHUNK = 80

EB = 8000
NB = 2000


def _bdot(a, b):
    return jnp.dot(a, b, preferred_element_type=jnp.float32)


def _edge_mlp_body(ef, We1, be1, We2, be2, e1o, e2o, se1o, se2o, acc1, acc2):
    i = pl.program_id(0)

    @pl.when(i == 0)
    def _():
        acc1[...] = jnp.zeros_like(acc1)
        acc2[...] = jnp.zeros_like(acc2)

    h = jnp.maximum(
        _bdot(ef[...], We1[...]) + be1[...], 0.0)
    e1o[...] = h
    z = jnp.maximum(
        _bdot(h, We2[...]) + be2[...], 0.0)
    e2o[...] = z
    acc1[...] += jnp.sum(h, axis=0, keepdims=True)
    acc2[...] += jnp.sum(z, axis=0, keepdims=True)

    @pl.when(i == pl.num_programs(0) - 1)
    def _():
        se1o[...] = acc1[...]
        se2o[...] = acc2[...]


def _edge_mlp(ef, We1, be1, We2, be2):
    grid = E // EB
    return pl.pallas_call(
        _edge_mlp_body,
        grid=(grid,),
        in_specs=[
            pl.BlockSpec((EB, D_EDGE), lambda i: (i, 0)),
            pl.BlockSpec((D_EDGE, D1), lambda i: (0, 0)),
            pl.BlockSpec((1, D1), lambda i: (0, 0)),
            pl.BlockSpec((D1, D2), lambda i: (0, 0)),
            pl.BlockSpec((1, D2), lambda i: (0, 0)),
        ],
        out_specs=[
            pl.BlockSpec((EB, D1), lambda i: (i, 0)),
            pl.BlockSpec((EB, D2), lambda i: (i, 0)),
            pl.BlockSpec((1, D1), lambda i: (0, 0)),
            pl.BlockSpec((1, D2), lambda i: (0, 0)),
        ],
        out_shape=[
            jax.ShapeDtypeStruct((E, D1), jnp.float32),
            jax.ShapeDtypeStruct((E, D2), jnp.float32),
            jax.ShapeDtypeStruct((1, D1), jnp.float32),
            jax.ShapeDtypeStruct((1, D2), jnp.float32),
        ],
        scratch_shapes=[
            pltpu.VMEM((1, D1), jnp.float32),
            pltpu.VMEM((1, D2), jnp.float32),
        ],
        compiler_params=pltpu.CompilerParams(
            dimension_semantics=("arbitrary",)),
    )(ef, We1, be1, We2, be2)


def _zero2d(ref, rows, width):
    zv = jnp.zeros((16,), jnp.float32)

    def row(r, _):
        def col(cc, _):
            ref[r, pl.ds(cc * 16, 16)] = zv
            return 0
        lax.fori_loop(0, width // 16, col, 0)
        return 0
    lax.fori_loop(0, rows, row, 0)


G = 4
NG_A = (E // NSUB) // CHUNK // G
REM_A = (E // NSUB) // CHUNK - NG_A * G
NG_B = EDGES_PER_TILE_B // CHUNK // G
REM_B = EDGES_PER_TILE_B // CHUNK - NG_B * G


def _sc_body(e1h, e2h, ridxh, sidxh, s1r, s2r, s1s, s2s, cro, cso, acc,
             b0, b1, b2, b3, i0, i1, i2, i3,
             g0, g1, g2, g3, s0, s1, s2, s3):
    cid = lax.axis_index("c")
    sid = lax.axis_index("s")
    fb1 = cid * F1
    tbase = sid * ROWS_MAIN
    nck = jnp.where(sid == NSUB - 1, ROWS_LAST // CHUNK, ROWS_MAIN // CHUNK)
    bufs = (b0, b1, b2, b3)
    idxs = (i0, i1, i2, i3)
    gsems = (g0, g1, g2, g3)
    ssems = (s0, s1, s2, s3)

    def zero_acc():
        _zero2d(bufs[0], CHUNK, F1)

        def zfire(k, _):
            pltpu.make_async_copy(
                bufs[0], acc.at[pl.ds(tbase + k * CHUNK, CHUNK)], gsems[0]).start()
            return 0
        lax.fori_loop(0, nck, zfire, 0)

        def zdrain(k, _):
            pltpu.make_async_copy(
                bufs[0], acc.at[pl.ds(tbase + k * CHUNK, CHUNK)], gsems[0]).wait()
            return 0
        lax.fori_loop(0, nck, zdrain, 0)

    def stream(ngrp, rem, gath, src=None):
        def scat(j):
            return pltpu.make_async_copy(
                bufs[j] if src is None else src, acc.at[idxs[j]], ssems[j])

        def grp(g, _):
            for j in range(G):
                @pl.when(g > 0)
                def _(j=j):
                    scat(j).wait()
                for d in gath(g * G + j, j):
                    d.start()
            for j in range(G):
                for d in gath(g * G + j, j):
                    d.wait()
                scat(j).start(add=True)
            return 0
        lax.fori_loop(0, ngrp, grp, 0)
        if ngrp > 0:
            for j in range(G):
                scat(j).wait()
        for r in range(rem):
            for d in gath(ngrp * G + r, r):
                d.start()
                d.wait()
            pltpu.sync_copy(bufs[r] if src is None else src,
                            acc.at[idxs[r]], add=True)

    def dump(dst):
        def dfire(k, _):
            r = tbase + k * CHUNK
            pltpu.make_async_copy(acc.at[pl.ds(r, CHUNK)], dst(r), gsems[0]).start()
            return 0
        lax.fori_loop(0, nck, dfire, 0)

        def ddrain(k, _):
            r = tbase + k * CHUNK
            pltpu.make_async_copy(acc.at[pl.ds(r, CHUNK)], dst(r), gsems[0]).wait()
            return 0
        lax.fori_loop(0, nck, ddrain, 0)

    def gath_a(idxh):
        def gath(c, j):
            base = sid * EDGES_PER_TILE + c * CHUNK
            return (pltpu.make_async_copy(idxh.at[pl.ds(base, CHUNK)],
                                          idxs[j], gsems[j]),
                    pltpu.make_async_copy(
                        e1h.at[pl.ds(base, CHUNK), pl.ds(fb1, F1)],
                        bufs[j], gsems[j]))
        return gath

    def gath_b(idxh):
        def gath(c, j):
            base = (cid * NSUB + sid) * EDGES_PER_TILE_B + c * CHUNK
            return (pltpu.make_async_copy(idxh.at[pl.ds(base, CHUNK)],
                                          idxs[j], gsems[j]),
                    pltpu.make_async_copy(e2h.at[pl.ds(base, CHUNK)],
                                          bufs[j], gsems[j]))
        return gath

    for idxh, s1o, s2o in ((ridxh, s1r, s2r), (sidxh, s1s, s2s)):
        zero_acc()
        plsc.subcore_barrier()
        stream(NG_A, REM_A, gath_a(idxh))
        plsc.subcore_barrier()
        dump(lambda r, s1o=s1o: s1o.at[pl.ds(r, CHUNK), pl.ds(fb1, F1)])

        zero_acc()
        plsc.subcore_barrier()
        stream(NG_B, REM_B, gath_b(idxh))
        plsc.subcore_barrier()
        dump(lambda r, s2o=s2o: s2o.at[pl.ds(cid * N + r, CHUNK)])

    zero_acc()
    ov = jnp.ones((16,), jnp.float32)

    def onerow(r, _):
        bufs[0][r, pl.ds(0, 16)] = ov
        return 0
    lax.fori_loop(0, CHUNK, onerow, 0)
    plsc.subcore_barrier()

    def gath_c(idxh):
        def gath(c, j):
            base = sid * EDGES_PER_TILE + c * CHUNK
            return (pltpu.make_async_copy(idxh.at[pl.ds(base, CHUNK)],
                                          idxs[j], gsems[j]),)
        return gath

    @pl.when(cid == 0)
    def _():
        stream(NG_A, REM_A, gath_c(ridxh), src=bufs[0])

    @pl.when(cid == 1)
    def _():
        stream(NG_A, REM_A, gath_c(sidxh), src=bufs[0])

    plsc.subcore_barrier()

    @pl.when(cid == 0)
    def _():
        dump(lambda r: cro.at[pl.ds(r, CHUNK)])

    @pl.when(cid == 1)
    def _():
        dump(lambda r: cso.at[pl.ds(r, CHUNK)])


@functools.cache
def _sc_kernel():
    return pl.kernel(
        _sc_body,
        out_type=[
            jax.ShapeDtypeStruct((N, D1), jnp.float32),
            jax.ShapeDtypeStruct((NCORES * N, D2), jnp.float32),
            jax.ShapeDtypeStruct((N, D1), jnp.float32),
            jax.ShapeDtypeStruct((NCORES * N, D2), jnp.float32),
            jax.ShapeDtypeStruct((N, 128), jnp.float32),
            jax.ShapeDtypeStruct((N, 128), jnp.float32),
        ],
        mesh=plsc.VectorSubcoreMesh(core_axis_name="c", subcore_axis_name="s",
                                    num_cores=NCORES, num_subcores=NSUB),
        scratch_types=(
            [pltpu.VMEM_SHARED((N, F1), jnp.float32)]
            + [pltpu.VMEM((CHUNK, F1), jnp.float32)] * G
            + [pltpu.VMEM((CHUNK,), jnp.int32)] * G
            + [pltpu.SemaphoreType.DMA] * (2 * G)
        ),
    )


def _node_mlp_body(nf, i1, o1, i2a, i2b, o2a, o2b, cr, cs, se1, se2, gf,
                   Wn1, Win1, Wout1, bn1, Wn2, Win2, Wout2, bn2,
                   Wg1, Wng1, Weg1, bg1, Wg2, Wng2, Weg2, bg2,
                   Wm, bm, Wls, bls, mo, lo, sn1, sn2):
    i = pl.program_id(0)

    @pl.when(i == 0)
    def _():
        sn1[...] = jnp.zeros_like(sn1)
        sn2[...] = jnp.zeros_like(sn2)

    crv = jnp.maximum(cr[:, 0:1], 1.0)
    csv = jnp.maximum(cs[:, 0:1], 1.0)
    dot = _bdot
    n1 = jnp.maximum(
        dot(nf[...], Wn1[...]) + dot(i1[...] / crv, Win1[...])
        + dot(o1[...] / csv, Wout1[...]) + bn1[...], 0.0)
    sn1[...] += jnp.sum(n1, axis=0, keepdims=True)
    n2 = jnp.maximum(
        dot(n1, Wn2[...]) + dot((i2a[...] + i2b[...]) / crv, Win2[...])
        + dot((o2a[...] + o2b[...]) / csv, Wout2[...]) + bn2[...], 0.0)
    sn2[...] += jnp.sum(n2, axis=0, keepdims=True)

    @pl.when(i == pl.num_programs(0) - 1)
    def _():
        g1 = jnp.maximum(
            dot(gf[...], Wg1[...]) + dot(sn1[...] / N, Wng1[...])
            + dot(se1[...] / E, Weg1[...]) + bg1[...], 0.0)
        g2 = jnp.maximum(
            dot(g1, Wg2[...]) + dot(sn2[...] / N, Wng2[...])
            + dot(se2[...] / E, Weg2[...]) + bg2[...], 0.0)
        mo[...] = dot(g2, Wm[...]) + bm[...]
        lo[...] = jnp.clip(dot(g2, Wls[...]) + bls[...],
                           LOG_SIG_MIN, LOG_SIG_MAX)


def _node_mlp(nf, i1, o1, i2a, i2b, o2a, o2b, cr, cs, se1, se2, gf, w):
    grid = N // NB
    full = lambda a, b: pl.BlockSpec((a, b), lambda i: (0, 0))
    return pl.pallas_call(
        _node_mlp_body,
        grid=(grid,),
        in_specs=[
            pl.BlockSpec((NB, D_NODE), lambda i: (i, 0)),
            pl.BlockSpec((NB, D1), lambda i: (i, 0)),
            pl.BlockSpec((NB, D1), lambda i: (i, 0)),
            pl.BlockSpec((NB, D2), lambda i: (i, 0)),
            pl.BlockSpec((NB, D2), lambda i: (i + N // NB, 0)),
            pl.BlockSpec((NB, D2), lambda i: (i, 0)),
            pl.BlockSpec((NB, D2), lambda i: (i + N // NB, 0)),
            pl.BlockSpec((NB, 128), lambda i: (i, 0)),
            pl.BlockSpec((NB, 128), lambda i: (i, 0)),
            full(1, D1), full(1, D2), full(1, D_GLOBAL := 16),
            full(D_NODE, D1), full(D1, D1), full(D1, D1), full(1, D1),
            full(D1, D2), full(D2, D2), full(D2, D2), full(1, D2),
            full(16, D1), full(D1, D1), full(D1, D1), full(1, D1),
            full(D1, D2), full(D2, D2), full(D2, D2), full(1, D2),
            full(D2, A), full(1, A), full(D2, A), full(1, A),
        ],
        out_specs=[
            pl.BlockSpec((1, A), lambda i: (0, 0)),
            pl.BlockSpec((1, A), lambda i: (0, 0)),
        ],
        out_shape=[
            jax.ShapeDtypeStruct((1, A), jnp.float32),
            jax.ShapeDtypeStruct((1, A), jnp.float32),
        ],
        scratch_shapes=[
            pltpu.VMEM((1, D1), jnp.float32),
            pltpu.VMEM((1, D2), jnp.float32),
        ],
        compiler_params=pltpu.CompilerParams(
            dimension_semantics=("arbitrary",)),
    )(nf, i1, o1, i2a, i2b, o2a, o2b, cr, cs, se1, se2, gf,
      w['Wn1'], w['Win1'], w['Wout1'], w['bn1'].reshape(1, D1),
      w['Wn2'], w['Win2'], w['Wout2'], w['bn2'].reshape(1, D2),
      w['Wg1'], w['Wng1'], w['Weg1'], w['bg1'].reshape(1, D1),
      w['Wg2'], w['Wng2'], w['Weg2'], w['bg2'].reshape(1, D2),
      w['Wm'], w['bm'].reshape(1, A), w['Wls'], w['bls'].reshape(1, A))


def _sc_scatter(e1, e2, receivers, senders):
    return _sc_kernel()(e1, e2, receivers, senders)


def kernel(node_features, edge_features, global_features, senders, receivers, params):
    p = params
    e1, e2, se1, se2 = _edge_mlp(
        edge_features, p['We1'], p['be1'].reshape(1, D1),
        p['We2'], p['be2'].reshape(1, D2))
    i1, i2p, o1, o2p, cr, cs = _sc_scatter(e1, e2, receivers, senders)
    mean, log_std = _node_mlp(node_features, i1, o1,
                              i2p, i2p, o2p, o2p, cr, cs,
                              se1, se2, global_features, params)
    return mean, log_std


# speedup vs baseline: 1.3260x; 1.0003x over previous
import functools

import jax
import jax.numpy as jnp
from jax import lax
from jax.experimental import pallas as pl
from jax.experimental.pallas import tpu as pltpu
from jax.experimental.pallas import tpu_sc as plsc

N = 10000
E = 320000
D_NODE = 128
D_EDGE = 16
D1 = 256
D2 = 128
A = 8
LOG_SIG_MAX = 2.0
LOG_SIG_MIN = -20.0

NCORES = 2
NSUB = 16
F1 = D1 // NCORES
CHUNK = 80
EDGES_PER_TILE = E // NSUB
EDGES_PER_TILE_B = E // (NCORES * NSUB)
ROWS_MAIN = 640
ROWS_LAST = N - (NSUB - 1) * ROWS_MAIN
CCHUNK = 80

EB = 10000
NB = 2000


def _bdot(a, b):
    return jnp.dot(a, b, preferred_element_type=jnp.float32)


def _edge_mlp_body(ef, We1, be1, We2, be2, e1o, e2o, se1o, se2o, acc1, acc2):
    i = pl.program_id(0)

    @pl.when(i == 0)
    def _():
        acc1[...] = jnp.zeros_like(acc1)
        acc2[...] = jnp.zeros_like(acc2)

    h = jnp.maximum(
        _bdot(ef[...], We1[...]) + be1[...], 0.0)
    e1o[...] = h
    z = jnp.maximum(
        _bdot(h, We2[...]) + be2[...], 0.0)
    e2o[...] = z
    acc1[...] += jnp.sum(h, axis=0, keepdims=True)
    acc2[...] += jnp.sum(z, axis=0, keepdims=True)

    @pl.when(i == pl.num_programs(0) - 1)
    def _():
        se1o[...] = acc1[...]
        se2o[...] = acc2[...]


def _edge_mlp(ef, We1, be1, We2, be2):
    grid = E // EB
    return pl.pallas_call(
        _edge_mlp_body,
        grid=(grid,),
        in_specs=[
            pl.BlockSpec((EB, D_EDGE), lambda i: (i, 0)),
            pl.BlockSpec((D_EDGE, D1), lambda i: (0, 0)),
            pl.BlockSpec((1, D1), lambda i: (0, 0)),
            pl.BlockSpec((D1, D2), lambda i: (0, 0)),
            pl.BlockSpec((1, D2), lambda i: (0, 0)),
        ],
        out_specs=[
            pl.BlockSpec((EB, D1), lambda i: (i, 0)),
            pl.BlockSpec((EB, D2), lambda i: (i, 0)),
            pl.BlockSpec((1, D1), lambda i: (0, 0)),
            pl.BlockSpec((1, D2), lambda i: (0, 0)),
        ],
        out_shape=[
            jax.ShapeDtypeStruct((E, D1), jnp.float32),
            jax.ShapeDtypeStruct((E, D2), jnp.float32),
            jax.ShapeDtypeStruct((1, D1), jnp.float32),
            jax.ShapeDtypeStruct((1, D2), jnp.float32),
        ],
        scratch_shapes=[
            pltpu.VMEM((1, D1), jnp.float32),
            pltpu.VMEM((1, D2), jnp.float32),
        ],
        compiler_params=pltpu.CompilerParams(
            dimension_semantics=("arbitrary",)),
    )(ef, We1, be1, We2, be2)


def _zero2d(ref, rows, width):
    zv = jnp.zeros((16,), jnp.float32)

    def row(r, _):
        def col(cc, _):
            ref[r, pl.ds(cc * 16, 16)] = zv
            return 0
        lax.fori_loop(0, width // 16, col, 0)
        return 0
    lax.fori_loop(0, rows, row, 0)


G = 4
NG_A = (E // NSUB) // CHUNK // G
REM_A = (E // NSUB) // CHUNK - NG_A * G
NG_B = EDGES_PER_TILE_B // CHUNK // G
REM_B = EDGES_PER_TILE_B // CHUNK - NG_B * G


def _sc_body(e1h, e2h, ridxh, sidxh, s1r, s2r, s1s, s2s, cro, cso, acc,
             b0, b1, b2, b3, i0, i1, i2, i3,
             g0, g1, g2, g3, s0, s1, s2, s3):
    cid = lax.axis_index("c")
    sid = lax.axis_index("s")
    fb1 = cid * F1
    tbase = sid * ROWS_MAIN
    nck = jnp.where(sid == NSUB - 1, ROWS_LAST // CHUNK, ROWS_MAIN // CHUNK)
    bufs = (b0, b1, b2, b3)
    idxs = (i0, i1, i2, i3)
    gsems = (g0, g1, g2, g3)
    ssems = (s0, s1, s2, s3)

    def zero_acc():
        _zero2d(bufs[0], CHUNK, F1)

        def zfire(k, _):
            pltpu.make_async_copy(
                bufs[0], acc.at[pl.ds(tbase + k * CHUNK, CHUNK)], gsems[0]).start()
            return 0
        lax.fori_loop(0, nck, zfire, 0)

        def zdrain(k, _):
            pltpu.make_async_copy(
                bufs[0], acc.at[pl.ds(tbase + k * CHUNK, CHUNK)], gsems[0]).wait()
            return 0
        lax.fori_loop(0, nck, zdrain, 0)

    def stream(ngrp, rem, gath, src=None):
        def scat(j):
            return pltpu.make_async_copy(
                bufs[j] if src is None else src, acc.at[idxs[j]], ssems[j])

        def grp(g, _):
            for j in range(G):
                @pl.when(g > 0)
                def _(j=j):
                    scat(j).wait()
                for d in gath(g * G + j, j):
                    d.start()
            for j in range(G):
                for d in gath(g * G + j, j):
                    d.wait()
                scat(j).start(add=True)
            return 0
        lax.fori_loop(0, ngrp, grp, 0)
        if ngrp > 0:
            for j in range(G):
                scat(j).wait()
        for r in range(rem):
            for d in gath(ngrp * G + r, r):
                d.start()
                d.wait()
            pltpu.sync_copy(bufs[r] if src is None else src,
                            acc.at[idxs[r]], add=True)

    def dump(dst):
        def dfire(k, _):
            r = tbase + k * CHUNK
            pltpu.make_async_copy(acc.at[pl.ds(r, CHUNK)], dst(r), gsems[0]).start()
            return 0
        lax.fori_loop(0, nck, dfire, 0)

        def ddrain(k, _):
            r = tbase + k * CHUNK
            pltpu.make_async_copy(acc.at[pl.ds(r, CHUNK)], dst(r), gsems[0]).wait()
            return 0
        lax.fori_loop(0, nck, ddrain, 0)

    def gath_a(idxh):
        def gath(c, j):
            base = sid * EDGES_PER_TILE + c * CHUNK
            return (pltpu.make_async_copy(idxh.at[pl.ds(base, CHUNK)],
                                          idxs[j], gsems[j]),
                    pltpu.make_async_copy(
                        e1h.at[pl.ds(base, CHUNK), pl.ds(fb1, F1)],
                        bufs[j], gsems[j]))
        return gath

    def gath_b(idxh):
        def gath(c, j):
            base = (cid * NSUB + sid) * EDGES_PER_TILE_B + c * CHUNK
            return (pltpu.make_async_copy(idxh.at[pl.ds(base, CHUNK)],
                                          idxs[j], gsems[j]),
                    pltpu.make_async_copy(e2h.at[pl.ds(base, CHUNK)],
                                          bufs[j], gsems[j]))
        return gath

    for idxh, s1o, s2o in ((ridxh, s1r, s2r), (sidxh, s1s, s2s)):
        zero_acc()
        plsc.subcore_barrier()
        stream(NG_A, REM_A, gath_a(idxh))
        plsc.subcore_barrier()
        dump(lambda r, s1o=s1o: s1o.at[pl.ds(r, CHUNK), pl.ds(fb1, F1)])

        zero_acc()
        plsc.subcore_barrier()
        stream(NG_B, REM_B, gath_b(idxh))
        plsc.subcore_barrier()
        dump(lambda r, s2o=s2o: s2o.at[pl.ds(cid * N + r, CHUNK)])

    zero_acc()
    ov = jnp.ones((16,), jnp.float32)

    def onerow(r, _):
        bufs[0][r, pl.ds(0, 16)] = ov
        return 0
    lax.fori_loop(0, CHUNK, onerow, 0)
    plsc.subcore_barrier()

    def gath_c(idxh):
        def gath(c, j):
            base = sid * EDGES_PER_TILE + c * CHUNK
            return (pltpu.make_async_copy(idxh.at[pl.ds(base, CHUNK)],
                                          idxs[j], gsems[j]),)
        return gath

    @pl.when(cid == 0)
    def _():
        stream(NG_A, REM_A, gath_c(ridxh), src=bufs[0])

    @pl.when(cid == 1)
    def _():
        stream(NG_A, REM_A, gath_c(sidxh), src=bufs[0])

    plsc.subcore_barrier()

    @pl.when(cid == 0)
    def _():
        dump(lambda r: cro.at[pl.ds(r, CHUNK)])

    @pl.when(cid == 1)
    def _():
        dump(lambda r: cso.at[pl.ds(r, CHUNK)])


@functools.cache
def _sc_kernel():
    return pl.kernel(
        _sc_body,
        out_type=[
            jax.ShapeDtypeStruct((N, D1), jnp.float32),
            jax.ShapeDtypeStruct((NCORES * N, D2), jnp.float32),
            jax.ShapeDtypeStruct((N, D1), jnp.float32),
            jax.ShapeDtypeStruct((NCORES * N, D2), jnp.float32),
            jax.ShapeDtypeStruct((N, 128), jnp.float32),
            jax.ShapeDtypeStruct((N, 128), jnp.float32),
        ],
        mesh=plsc.VectorSubcoreMesh(core_axis_name="c", subcore_axis_name="s",
                                    num_cores=NCORES, num_subcores=NSUB),
        scratch_types=(
            [pltpu.VMEM_SHARED((N, F1), jnp.float32)]
            + [pltpu.VMEM((CHUNK, F1), jnp.float32)] * G
            + [pltpu.VMEM((CHUNK,), jnp.int32)] * G
            + [pltpu.SemaphoreType.DMA] * (2 * G)
        ),
    )


def _node_mlp_body(nf, i1, o1, i2a, i2b, o2a, o2b, cr, cs, se1, se2, gf,
                   Wn1, Win1, Wout1, bn1, Wn2, Win2, Wout2, bn2,
                   Wg1, Wng1, Weg1, bg1, Wg2, Wng2, Weg2, bg2,
                   Wm, bm, Wls, bls, mo, lo, sn1, sn2):
    i = pl.program_id(0)

    @pl.when(i == 0)
    def _():
        sn1[...] = jnp.zeros_like(sn1)
        sn2[...] = jnp.zeros_like(sn2)

    crv = jnp.maximum(cr[:, 0:1], 1.0)
    csv = jnp.maximum(cs[:, 0:1], 1.0)
    dot = _bdot
    n1 = jnp.maximum(
        dot(nf[...], Wn1[...]) + dot(i1[...] / crv, Win1[...])
        + dot(o1[...] / csv, Wout1[...]) + bn1[...], 0.0)
    sn1[...] += jnp.sum(n1, axis=0, keepdims=True)
    n2 = jnp.maximum(
        dot(n1, Wn2[...]) + dot((i2a[...] + i2b[...]) / crv, Win2[...])
        + dot((o2a[...] + o2b[...]) / csv, Wout2[...]) + bn2[...], 0.0)
    sn2[...] += jnp.sum(n2, axis=0, keepdims=True)

    @pl.when(i == pl.num_programs(0) - 1)
    def _():
        g1 = jnp.maximum(
            dot(gf[...], Wg1[...]) + dot(sn1[...] / N, Wng1[...])
            + dot(se1[...] / E, Weg1[...]) + bg1[...], 0.0)
        g2 = jnp.maximum(
            dot(g1, Wg2[...]) + dot(sn2[...] / N, Wng2[...])
            + dot(se2[...] / E, Weg2[...]) + bg2[...], 0.0)
        mo[...] = dot(g2, Wm[...]) + bm[...]
        lo[...] = jnp.clip(dot(g2, Wls[...]) + bls[...],
                           LOG_SIG_MIN, LOG_SIG_MAX)


def _node_mlp(nf, i1, o1, i2a, i2b, o2a, o2b, cr, cs, se1, se2, gf, w):
    grid = N // NB
    full = lambda a, b: pl.BlockSpec((a, b), lambda i: (0, 0))
    return pl.pallas_call(
        _node_mlp_body,
        grid=(grid,),
        in_specs=[
            pl.BlockSpec((NB, D_NODE), lambda i: (i, 0)),
            pl.BlockSpec((NB, D1), lambda i: (i, 0)),
            pl.BlockSpec((NB, D1), lambda i: (i, 0)),
            pl.BlockSpec((NB, D2), lambda i: (i, 0)),
            pl.BlockSpec((NB, D2), lambda i: (i + N // NB, 0)),
            pl.BlockSpec((NB, D2), lambda i: (i, 0)),
            pl.BlockSpec((NB, D2), lambda i: (i + N // NB, 0)),
            pl.BlockSpec((NB, 128), lambda i: (i, 0)),
            pl.BlockSpec((NB, 128), lambda i: (i, 0)),
            full(1, D1), full(1, D2), full(1, D_GLOBAL := 16),
            full(D_NODE, D1), full(D1, D1), full(D1, D1), full(1, D1),
            full(D1, D2), full(D2, D2), full(D2, D2), full(1, D2),
            full(16, D1), full(D1, D1), full(D1, D1), full(1, D1),
            full(D1, D2), full(D2, D2), full(D2, D2), full(1, D2),
            full(D2, A), full(1, A), full(D2, A), full(1, A),
        ],
        out_specs=[
            pl.BlockSpec((1, A), lambda i: (0, 0)),
            pl.BlockSpec((1, A), lambda i: (0, 0)),
        ],
        out_shape=[
            jax.ShapeDtypeStruct((1, A), jnp.float32),
            jax.ShapeDtypeStruct((1, A), jnp.float32),
        ],
        scratch_shapes=[
            pltpu.VMEM((1, D1), jnp.float32),
            pltpu.VMEM((1, D2), jnp.float32),
        ],
        compiler_params=pltpu.CompilerParams(
            dimension_semantics=("arbitrary",)),
    )(nf, i1, o1, i2a, i2b, o2a, o2b, cr, cs, se1, se2, gf,
      w['Wn1'], w['Win1'], w['Wout1'], w['bn1'].reshape(1, D1),
      w['Wn2'], w['Win2'], w['Wout2'], w['bn2'].reshape(1, D2),
      w['Wg1'], w['Wng1'], w['Weg1'], w['bg1'].reshape(1, D1),
      w['Wg2'], w['Wng2'], w['Weg2'], w['bg2'].reshape(1, D2),
      w['Wm'], w['bm'].reshape(1, A), w['Wls'], w['bls'].reshape(1, A))


def _sc_scatter(e1, e2, receivers, senders):
    return _sc_kernel()(e1, e2, receivers, senders)


def kernel(node_features, edge_features, global_features, senders, receivers, params):
    p = params
    e1, e2, se1, se2 = _edge_mlp(
        edge_features, p['We1'], p['be1'].reshape(1, D1),
        p['We2'], p['be2'].reshape(1, D2))
    i1, i2p, o1, o2p, cr, cs = _sc_scatter(e1, e2, receivers, senders)
    mean, log_std = _node_mlp(node_features, i1, o1,
                              i2p, i2p, o2p, o2p, cr, cs,
                              se1, se2, global_features, params)
    return mean, log_std
